# baseline (speedup 1.0000x reference)
import functools

import jax
import jax.numpy as jnp
from jax import lax
from jax.experimental import pallas as pl
from jax.experimental.pallas import tpu as pltpu
from jax.experimental.pallas import tpu_sc as plsc

EMBED = 768
MAX_POS = 64
LANES = 16


@functools.cache
def _make_kernel(n_rows):
    info = plsc.get_sparse_core_info()
    nc, ns = info.num_cores, info.num_subcores
    nw = nc * ns
    rows_per_w = n_rows // nw
    chunk = MAX_POS
    n_chunks = rows_per_w // chunk
    mesh = plsc.VectorSubcoreMesh(core_axis_name="c", subcore_axis_name="s")

    @functools.partial(
        pl.kernel,
        out_type=jax.ShapeDtypeStruct((n_rows, EMBED), jnp.float32),
        mesh=mesh,
        scratch_types=[
            pltpu.VMEM((chunk,), jnp.int32),
            pltpu.VMEM((chunk, EMBED), jnp.float32),
            pltpu.VMEM((MAX_POS, EMBED), jnp.float32),
            pltpu.SemaphoreType.DMA,
        ],
    )
    def k(ids_hbm, table_hbm, pos_hbm, out_hbm, idx_v, rows_v, pos_v, sem):
        wid = lax.axis_index("s") * nc + lax.axis_index("c")
        base = wid * rows_per_w
        pltpu.sync_copy(pos_hbm, pos_v)

        def chunk_body(ci, carry):
            row0 = base + ci * chunk
            pltpu.sync_copy(ids_hbm.at[pl.ds(row0, chunk)], idx_v)
            pltpu.async_copy(table_hbm.at[idx_v], rows_v, sem).wait()

            def add_row(r, c2):
                def add_vec(d, c3):
                    col = d * LANES
                    rows_v[r, pl.ds(col, LANES)] = (
                        rows_v[r, pl.ds(col, LANES)]
                        + pos_v[r, pl.ds(col, LANES)]
                    )
                    return c3

                return lax.fori_loop(0, EMBED // LANES, add_vec, c2)

            lax.fori_loop(0, chunk, add_row, 0)
            pltpu.sync_copy(rows_v, out_hbm.at[pl.ds(row0, chunk)])
            return carry

        lax.fori_loop(0, n_chunks, chunk_body, 0)

    return k


def kernel(input_ids, token_embedding, position_embedding):
    b, s = input_ids.shape
    ids_flat = input_ids.reshape(b * s).astype(jnp.int32)
    out = _make_kernel(b * s)(ids_flat, token_embedding, position_embedding)
    return out.reshape(b, s, EMBED)


# speedup vs baseline: 4.2754x; 4.2754x over previous
import functools

import jax
import jax.numpy as jnp
from jax import lax
from jax.experimental import pallas as pl
from jax.experimental.pallas import tpu as pltpu
from jax.experimental.pallas import tpu_sc as plsc

EMBED = 768
MAX_POS = 64
LANES = 16
CHUNK = 32


@functools.cache
def _make_kernel(n_rows):
    info = plsc.get_sparse_core_info()
    nc, ns = info.num_cores, info.num_subcores
    nw = nc * ns
    rows_per_w = n_rows // nw
    n_chunks = rows_per_w // CHUNK
    n_pairs = n_chunks // 2
    mesh = plsc.VectorSubcoreMesh(core_axis_name="c", subcore_axis_name="s")

    @functools.partial(
        pl.kernel,
        out_type=jax.ShapeDtypeStruct((n_rows, EMBED), jnp.float32),
        mesh=mesh,
        scratch_types=[
            pltpu.VMEM((CHUNK,), jnp.int32),
            pltpu.VMEM((CHUNK,), jnp.int32),
            pltpu.VMEM((CHUNK, EMBED), jnp.float32),
            pltpu.VMEM((CHUNK, EMBED), jnp.float32),
            pltpu.VMEM((MAX_POS, EMBED), jnp.float32),
            pltpu.VMEM((CHUNK,), jnp.int32),
            pltpu.SemaphoreType.DMA,
            pltpu.SemaphoreType.DMA,
        ],
    )
    def k(ids_hbm, table_hbm, pos_hbm, out_hbm,
          idx0_v, idx1_v, rows0_v, rows1_v, pos_v, iota_v, sem0, sem1):
        wid = lax.axis_index("s") * nc + lax.axis_index("c")
        base = wid * rows_per_w
        pltpu.sync_copy(pos_hbm, pos_v)

        def issue(c, idx_v, rows_v, sem):
            pltpu.sync_copy(ids_hbm.at[pl.ds(c, CHUNK)], idx_v)
            pltpu.async_copy(table_hbm.at[idx_v], rows_v, sem)

        def finish(c, idx_v, rows_v, sem, pos_off):
            pltpu.make_async_copy(table_hbm.at[idx_v], rows_v, sem).wait()

            @functools.partial(plsc.parallel_loop, 0, CHUNK)
            def _(r):
                for d in range(EMBED // LANES):
                    col = d * LANES
                    rows_v[r, pl.ds(col, LANES)] = (
                        rows_v[r, pl.ds(col, LANES)]
                        + pos_v[pos_off + r, pl.ds(col, LANES)]
                    )

            pltpu.sync_copy(rows_v, out_hbm.at[pl.ds(c, CHUNK)])

        issue(base, idx0_v, rows0_v, sem0)

        def pair_body(i, carry):
            c0 = base + (2 * i) * CHUNK
            c1 = c0 + CHUNK
            issue(c1, idx1_v, rows1_v, sem1)
            finish(c0, idx0_v, rows0_v, sem0, 0)

            @pl.when(i + 1 < n_pairs)
            def _():
                issue(c1 + CHUNK, idx0_v, rows0_v, sem0)

            finish(c1, idx1_v, rows1_v, sem1, CHUNK)
            return carry

        lax.fori_loop(0, n_pairs, pair_body, 0)

    return k


def kernel(input_ids, token_embedding, position_embedding):
    b, s = input_ids.shape
    ids_flat = input_ids.reshape(b * s).astype(jnp.int32)
    out = _make_kernel(b * s)(ids_flat, token_embedding, position_embedding)
    return out.reshape(b, s, EMBED)
